# baseline (speedup 1.0000x reference)
import functools

import jax
import jax.numpy as jnp
from jax import lax
from jax.experimental import pallas as pl
from jax.experimental.pallas import tpu as pltpu
from jax.experimental.pallas import tpu_sc as plsc

NC = 2
NS = 16
NW = NC * NS
L = 16

B = 16384
D = 64
V = 1000000
BW = B // NW
NCHUNK = BW // L

TROW = ((V + 127) // 128) * 1024


def _rsqrt_nr(x):
    i = lax.bitcast_convert_type(x, jnp.int32)
    y = lax.bitcast_convert_type(jnp.int32(0x5F3759DF) - (i >> 1), jnp.float32)
    half_x = 0.5 * x
    for _ in range(3):
        y = y * (1.5 - half_x * y * y)
    return y


def _pow34(x):
    s = x * _rsqrt_nr(x)
    q = s * _rsqrt_nr(s)
    return s * q


def _k1_body(wi_hbm, wj_hbm, wt_hbm, wc_hbm, yp_hbm, pw_hbm,
             idxi_v, idxj_v, ei_v, ej_v, yp_v, pw_v, sem):
    wid = lax.axis_index("s") * NC + lax.axis_index("c")

    pltpu.sync_copy(wi_hbm.at[wid], idxi_v)
    pltpu.sync_copy(wj_hbm.at[wid], idxj_v)

    pending = []
    for dg in range(D // 8):
        group = []
        for d in range(dg * 8, dg * 8 + 8):
            group.append(pltpu.async_copy(
                wt_hbm.at[d].at[idxi_v], ei_v.at[d], sem))
            group.append(pltpu.async_copy(
                wc_hbm.at[d].at[idxj_v], ej_v.at[d], sem))
        pending.append(group)
        if len(pending) > 2:
            for cp in pending.pop(0):
                cp.wait()
    for group in pending:
        for cp in group:
            cp.wait()

    def group_body(g, carry):
        off = pl.ds(g * L, L)
        acc = jnp.zeros((L,), jnp.float32)
        for d in range(D):
            acc = acc + ei_v[d, off] * ej_v[d, off]
        yp_v[off] = acc
        return carry

    lax.fori_loop(0, BW // L, group_body, jnp.int32(0))

    pltpu.sync_copy(yp_v, yp_hbm.at[pl.ds(wid * BW, BW)])

    def pow_body(v, acc):
        x = yp_v[pl.ds(v * L, L)] / jnp.float32(100.0)
        return acc + _pow34(x)

    acc = lax.fori_loop(0, BW // L, pow_body, jnp.zeros((L,), jnp.float32))
    pw_v[...] = acc
    pltpu.sync_copy(pw_v, pw_hbm.at[wid])


@functools.lru_cache(maxsize=1)
def _get_k1():
    return pl.kernel(
        _k1_body,
        out_type=[
            jax.ShapeDtypeStruct((B,), jnp.float32),
            jax.ShapeDtypeStruct((NW, L), jnp.float32),
        ],
        mesh=plsc.VectorSubcoreMesh(core_axis_name="c", subcore_axis_name="s"),
        compiler_params=pltpu.CompilerParams(
            needs_layout_passes=False, use_tc_tiling_on_sc=False,
            disable_bounds_checks=True),
        scratch_types=[
            pltpu.VMEM((BW,), jnp.int32),
            pltpu.VMEM((BW,), jnp.int32),
            pltpu.VMEM((D, BW), jnp.float32),
            pltpu.VMEM((D, BW), jnp.float32),
            pltpu.VMEM((BW,), jnp.float32),
            pltpu.VMEM((L,), jnp.float32),
            pltpu.SemaphoreType.DMA,
        ],
    )


def _k2_body(yp_ref, yt_ref, pw_ref, o_ref):
    ws = jnp.sum(pw_ref[...])
    d = yp_ref[...] - jnp.log(yt_ref[...].astype(jnp.float32))
    o_ref[...] = ws * (d * d)


def kernel(w_i, w_j, y_true, W_target, W_context):
    wi2 = w_i.reshape(NW, BW)
    wj2 = w_j.reshape(NW, BW)
    ypred, partials = _get_k1()(wi2, wj2, W_target.T, W_context.T)
    out2d = pl.pallas_call(
        _k2_body,
        out_shape=jax.ShapeDtypeStruct((128, 128), jnp.float32),
    )(ypred.reshape(128, 128), y_true.reshape(128, 128), partials)
    return out2d.reshape(B, 1)


# speedup vs baseline: 8.8100x; 8.8100x over previous
import functools

import jax
import jax.numpy as jnp
from jax import lax
from jax.experimental import pallas as pl
from jax.experimental.pallas import tpu as pltpu
from jax.experimental.pallas import tpu_sc as plsc

NC = 2
NS = 16
NW = NC * NS

B = 16384
D = 64
BW = B // NW
CH = 4
CW = BW // CH


def _k1_body(wi_hbm, wj_hbm, wt_hbm, wc_hbm, ei_hbm, ej_hbm,
             idxi_v, idxj_v, ei_v, ej_v, sem):
    wid = lax.axis_index("s") * NC + lax.axis_index("c")

    pltpu.sync_copy(wi_hbm.at[wid], idxi_v)
    pltpu.sync_copy(wj_hbm.at[wid], idxj_v)

    cps = []
    for j in range(CH):
        dst = pl.ds(j * CW, CW)
        cps.append(pltpu.async_copy(wt_hbm.at[idxi_v.at[j]], ei_v.at[dst], sem))
        cps.append(pltpu.async_copy(wc_hbm.at[idxj_v.at[j]], ej_v.at[dst], sem))
    for cp in cps:
        cp.wait()

    pltpu.sync_copy(ei_v, ei_hbm.at[pl.ds(wid * BW, BW)])
    pltpu.sync_copy(ej_v, ej_hbm.at[pl.ds(wid * BW, BW)])


@functools.lru_cache(maxsize=1)
def _get_k1():
    return pl.kernel(
        _k1_body,
        out_type=[
            jax.ShapeDtypeStruct((B, D), jnp.float32),
            jax.ShapeDtypeStruct((B, D), jnp.float32),
        ],
        mesh=plsc.VectorSubcoreMesh(core_axis_name="c", subcore_axis_name="s"),
        compiler_params=pltpu.CompilerParams(
            needs_layout_passes=False, use_tc_tiling_on_sc=False,
            disable_bounds_checks=True),
        scratch_types=[
            pltpu.VMEM((CH, CW), jnp.int32),
            pltpu.VMEM((CH, CW), jnp.int32),
            pltpu.VMEM((BW, D), jnp.float32),
            pltpu.VMEM((BW, D), jnp.float32),
            pltpu.SemaphoreType.DMA,
        ],
    )


def _k2_body(ei_ref, ej_ref, yt_ref, o_ref):
    yp = jnp.sum(ei_ref[...] * ej_ref[...], axis=1, keepdims=True)
    ws = jnp.sum(jnp.power(yp * jnp.float32(0.01), jnp.float32(0.75)))
    d = yp - jnp.log(yt_ref[...].astype(jnp.float32))
    o_ref[...] = ws * (d * d)


def kernel(w_i, w_j, y_true, W_target, W_context):
    wi3 = w_i.reshape(NW, CH, CW)
    wj3 = w_j.reshape(NW, CH, CW)
    ei, ej = _get_k1()(wi3, wj3, W_target, W_context)
    return pl.pallas_call(
        _k2_body,
        out_shape=jax.ShapeDtypeStruct((B, 1), jnp.float32),
    )(ei, ej, y_true)
